# Initial kernel scaffold; baseline (speedup 1.0000x reference)
import jax
import jax.numpy as jnp
from jax.experimental import pallas as pl


def kernel(preds, gt_masks):
    raise NotImplementedError("write your pallas kernel here")


# speedup vs baseline: 16.6480x; 16.6480x over previous
import jax
import jax.numpy as jnp
from jax.experimental import pallas as pl
from jax.experimental.pallas import tpu as pltpu

R, C = 4096, 1024
BR = 512
NB = R // BR
N_PIX = R * C
K_COUNT = int(N_PIX * 10 / 100)
EPSILON = 3.1
NPASS = 16


def _k1_body(p_ref, g_ref, q_ref, sums_ref, acc_ref):
    b = pl.program_id(0)

    @pl.when(b == 0)
    def _():
        acc_ref[0] = 0.0
        acc_ref[1] = 0.0
        acc_ref[2] = 0.0

    p = p_ref[...]
    g = g_ref[...]
    q = jnp.where(g == 1.0, p, 1.0 - p)
    q_ref[...] = q
    acc_ref[0] += jnp.sum(p * g)
    acc_ref[1] += jnp.sum(p)
    acc_ref[2] += jnp.sum(g)

    @pl.when(b == NB - 1)
    def _():
        row = jax.lax.broadcasted_iota(jnp.int32, (8, 128), 0)
        lane = jax.lax.broadcasted_iota(jnp.int32, (8, 128), 1)
        v = jnp.where((row == 0) & (lane == 0), acc_ref[0], 0.0)
        v = jnp.where((row == 0) & (lane == 1), acc_ref[1], v)
        v = jnp.where((row == 0) & (lane == 2), acc_ref[2], v)
        sums_ref[...] = v


def _k2_body(sums_ref, q_ref, out_ref, st_ref, fa_ref):
    p = pl.program_id(0)
    b = pl.program_id(1)

    @pl.when((p == 0) & (b == 0))
    def _():
        st_ref[0] = 0
        st_ref[1] = K_COUNT
        st_ref[2] = 0
        fa_ref[0] = 0.0

    @pl.when(b == 0)
    def _():
        st_ref[3] = 0
        st_ref[4] = 0
        st_ref[5] = 0

    q = q_ref[...]
    bits = jax.lax.bitcast_convert_type(q, jnp.int32)

    @pl.when(p < NPASS)
    def _():
        shift = 30 - 2 * p
        hi_sh = jnp.minimum(shift + 2, 31)
        mask_in = jax.lax.shift_right_logical(bits, hi_sh) == st_ref[0]
        field = jax.lax.shift_right_logical(bits, shift) & 3
        st_ref[3] += jnp.sum(jnp.where(mask_in & (field == 0), 1, 0))
        st_ref[4] += jnp.sum(jnp.where(mask_in & (field == 1), 1, 0))
        st_ref[5] += jnp.sum(jnp.where(mask_in & (field == 2), 1, 0))

        @pl.when(b == NB - 1)
        def _():
            c0 = st_ref[3]
            c1 = st_ref[4]
            c2 = st_ref[5]
            r = st_ref[1]
            j = ((r > c0).astype(jnp.int32)
                 + (r > c0 + c1).astype(jnp.int32)
                 + (r > c0 + c1 + c2).astype(jnp.int32))
            below = (jnp.where(j >= 1, c0, 0)
                     + jnp.where(j >= 2, c1, 0)
                     + jnp.where(j >= 3, c2, 0))
            st_ref[0] = (st_ref[0] << 2) | j
            st_ref[1] = r - below
            st_ref[2] = st_ref[2] + below

    @pl.when(p == NPASS)
    def _():
        tbits = st_ref[0]
        lq = jnp.maximum(jnp.log(q), -100.0)
        bce = -lq
        poly = bce + (1.0 - jnp.exp(-bce)) * EPSILON
        fa_ref[0] += jnp.sum(jnp.where(bits < tbits, poly, 0.0))

        @pl.when(b == NB - 1)
        def _():
            row = jax.lax.broadcasted_iota(jnp.int32, (8, 128), 0)
            lane = jax.lax.broadcasted_iota(jnp.int32, (8, 128), 1)
            s = sums_ref[...]
            inter = jnp.sum(jnp.where((row == 0) & (lane == 0), s, 0.0))
            sum_p = jnp.sum(jnp.where((row == 0) & (lane == 1), s, 0.0))
            sum_g = jnp.sum(jnp.where((row == 0) & (lane == 2), s, 0.0))
            dice = 1.0 - (2.0 * inter + 1.0) / (sum_p + sum_g + 1.0)

            tq = jax.lax.bitcast_convert_type(
                jnp.full((8, 128), tbits, jnp.int32), jnp.float32)
            bce_t = -jnp.maximum(jnp.log(tq), -100.0)
            poly_t = bce_t + (1.0 - jnp.exp(-bce_t)) * EPSILON
            n_less = st_ref[2]
            total = fa_ref[0] + (K_COUNT - n_less).astype(jnp.float32) * poly_t
            out_ref[...] = dice + total / jnp.float32(K_COUNT)


def kernel(preds, gt_masks):
    p2 = preds.reshape(R, C)
    g2 = gt_masks.reshape(R, C)

    q, sums = pl.pallas_call(
        _k1_body,
        grid=(NB,),
        in_specs=[
            pl.BlockSpec((BR, C), lambda b: (b, 0)),
            pl.BlockSpec((BR, C), lambda b: (b, 0)),
        ],
        out_specs=[
            pl.BlockSpec((BR, C), lambda b: (b, 0)),
            pl.BlockSpec((8, 128), lambda b: (0, 0)),
        ],
        out_shape=[
            jax.ShapeDtypeStruct((R, C), jnp.float32),
            jax.ShapeDtypeStruct((8, 128), jnp.float32),
        ],
        scratch_shapes=[pltpu.SMEM((4,), jnp.float32)],
        compiler_params=pltpu.CompilerParams(
            dimension_semantics=("arbitrary",)),
    )(p2, g2)

    out = pl.pallas_call(
        _k2_body,
        grid=(NPASS + 1, NB),
        in_specs=[
            pl.BlockSpec((8, 128), lambda p, b: (0, 0)),
            pl.BlockSpec((BR, C), lambda p, b: (b, 0)),
        ],
        out_specs=pl.BlockSpec((8, 128), lambda p, b: (0, 0)),
        out_shape=jax.ShapeDtypeStruct((8, 128), jnp.float32),
        scratch_shapes=[
            pltpu.SMEM((8,), jnp.int32),
            pltpu.SMEM((2,), jnp.float32),
        ],
        compiler_params=pltpu.CompilerParams(
            dimension_semantics=("arbitrary", "arbitrary")),
    )(sums, q)

    return out[0, 0]
